# baseline (speedup 1.0000x reference)
import jax
import jax.numpy as jnp
from jax import lax
from jax.experimental import pallas as pl
from jax.experimental.pallas import tpu as pltpu
from jax.experimental.pallas import tpu_sc as plsc

VOCAB = 100000
DIM = 64
MAX_POS = 512
B = 1024
L = 200

NC, NS, LANES = 2, 16, 16
NW = NC * NS
SEQ_PER_W = B // NW
SEQ_PER_CHUNK = 2
CHUNK = SEQ_PER_CHUNK * L
NCHUNK = SEQ_PER_W // SEQ_PER_CHUNK
SUB = 40
NSUB = L // SUB


def _rsqrt16(v):
    i = lax.bitcast_convert_type(v, jnp.int32)
    i = jnp.int32(0x5F3759DF) - (i >> 1)
    y = lax.bitcast_convert_type(i, jnp.float32)
    for _ in range(2):
        y = y * (1.5 - 0.5 * v * y * y)
    return y


def _body(ids_hbm, weight_hbm, pos_hbm, gamma_hbm, beta_hbm, out_hbm,
          idx_v, in_v, stage_v, pos_v, gamma_v, beta_v, gsem, wsem):
    wid = lax.axis_index("s") * NC + lax.axis_index("c")

    pltpu.sync_copy(ids_hbm.at[pl.ds(wid * SEQ_PER_W, SEQ_PER_W)], idx_v)
    pltpu.sync_copy(pos_hbm.at[pl.ds(0, L)], pos_v)
    pltpu.sync_copy(gamma_hbm, gamma_v)
    pltpu.sync_copy(beta_hbm, beta_v)

    gvec = [gamma_v[pl.ds(k * LANES, LANES)] for k in range(DIM // LANES)]
    bvec = [beta_v[pl.ds(k * LANES, LANES)] for k in range(DIM // LANES)]

    def issue_gather(c, bi):
        for sl in range(SEQ_PER_CHUNK):
            for k in range(NSUB):
                pltpu.async_copy(
                    weight_hbm.at[idx_v.at[c * SEQ_PER_CHUNK + sl,
                                           pl.ds(k * SUB, SUB)]],
                    in_v.at[bi].at[pl.ds(sl * L + k * SUB, SUB)],
                    gsem.at[bi])

    def compute_chunk(bi):
        in_ref = in_v.at[bi]
        out_ref = stage_v.at[bi]

        @plsc.parallel_loop(0, L, unroll=4)
        def lbody(l):
            p = [pos_v[l, pl.ds(k * LANES, LANES)] for k in range(DIM // LANES)]
            for s_local in range(SEQ_PER_CHUNK):
                t = l + s_local * L
                x = [in_ref[t, pl.ds(k * LANES, LANES)] + p[k]
                     for k in range(DIM // LANES)]
                s4 = (x[0] + x[1]) + (x[2] + x[3])
                q4 = ((x[0] * x[0] + x[1] * x[1])
                      + (x[2] * x[2] + x[3] * x[3]))
                mean = jnp.sum(s4) * (1.0 / DIM)
                var = jnp.maximum(jnp.sum(q4) * (1.0 / DIM) - mean * mean,
                                  0.0) + 1e-12
                r = _rsqrt16(jnp.full((LANES,), var))
                mean_v = jnp.full((LANES,), mean)
                for k in range(DIM // LANES):
                    y = (x[k] - mean_v) * (r * gvec[k]) + bvec[k]
                    out_ref[s_local, l, pl.ds(k * LANES, LANES)] = y

    def wait_gather(bi):
        pltpu.make_async_copy(weight_hbm.at[pl.ds(0, CHUNK)], in_v.at[bi],
                              gsem.at[bi]).wait()

    def wait_wb(bi):
        pltpu.make_async_copy(stage_v.at[bi],
                              out_hbm.at[pl.ds(0, SEQ_PER_CHUNK)],
                              wsem.at[bi]).wait()

    issue_gather(0, 0)
    issue_gather(1, 1)

    def chunk_body(c, _):
        bi = c % 2
        wait_gather(bi)

        @pl.when(c >= 2)
        def _():
            wait_wb(bi)

        compute_chunk(bi)
        off = pl.multiple_of(wid * SEQ_PER_W + c * SEQ_PER_CHUNK,
                             SEQ_PER_CHUNK)
        pltpu.async_copy(stage_v.at[bi],
                         out_hbm.at[pl.ds(off, SEQ_PER_CHUNK)],
                         wsem.at[bi])

        @pl.when(c + 2 < NCHUNK)
        def _():
            issue_gather(c + 2, bi)

        return 0

    lax.fori_loop(0, NCHUNK, chunk_body, 0)
    wait_wb(0)
    wait_wb(1)


@jax.jit
def kernel(input_ids, weight, position_embeddings, gamma, beta):
    run = pl.kernel(
        _body,
        out_type=jax.ShapeDtypeStruct((B, L, DIM), jnp.float32),
        mesh=plsc.VectorSubcoreMesh(core_axis_name="c", subcore_axis_name="s"),
        compiler_params=pltpu.CompilerParams(needs_layout_passes=False,
                                             use_tc_tiling_on_sc=False),
        scratch_types=[
            pltpu.VMEM((SEQ_PER_W, L), jnp.int32),
            pltpu.VMEM((2, CHUNK, DIM), jnp.float32),
            pltpu.VMEM((2, SEQ_PER_CHUNK, L, DIM), jnp.float32),
            pltpu.VMEM((L, DIM), jnp.float32),
            pltpu.VMEM((DIM,), jnp.float32),
            pltpu.VMEM((DIM,), jnp.float32),
            pltpu.SemaphoreType.DMA((2,)),
            pltpu.SemaphoreType.DMA((2,)),
        ],
    )
    return run(input_ids, weight, position_embeddings, gamma, beta)
